# Initial kernel scaffold; baseline (speedup 1.0000x reference)
import jax
import jax.numpy as jnp
from jax.experimental import pallas as pl


def kernel(s_t, v_t, W_enc, permanences, W_l6b, b_l6b):
    raise NotImplementedError("write your pallas kernel here")


# speedup vs baseline: 1.1248x; 1.1248x over previous
import jax
import jax.numpy as jnp
import numpy as np
from jax import lax
from jax.experimental import pallas as pl
from jax.experimental.pallas import tpu as pltpu

_N_COLUMNS = 4
_INPUT_DIM = 1024
_N_SDR = 8192
_W_SPARSITY = 164
_N_MINI = 2048
_K_ACTIVE = 80
_IDX_PAD = 256

_INTERPRET = False


def _act_body(s_ref, w_ref, o_ref):
    o_ref[...] = lax.dot_general(
        s_ref[...], w_ref[...], (((1,), (1,)), ((), ())),
        preferred_element_type=jnp.float32)[None]


def _stage_a(w2d, s_row):
    br = 1024
    n = w2d.shape[0] // br
    return pl.pallas_call(
        _act_body,
        grid=(n,),
        in_specs=[
            pl.BlockSpec((1, _INPUT_DIM), lambda i: (0, 0)),
            pl.BlockSpec((br, _INPUT_DIM), lambda i: (i, 0)),
        ],
        out_specs=pl.BlockSpec((1, 1, br), lambda i: (i, 0, 0)),
        out_shape=jax.ShapeDtypeStruct((n, 1, br), jnp.float32),
        interpret=_INTERPRET,
    )(s_row, w2d)


def _cumsum8192(x):
    x2 = x.reshape(64, 128)
    ii = lax.broadcasted_iota(jnp.int32, (128, 128), 0)
    jj = lax.broadcasted_iota(jnp.int32, (128, 128), 1)
    li = (ii <= jj).astype(jnp.float32)
    cs = jnp.dot(x2, li, preferred_element_type=jnp.float32)
    rt = cs[:, 127].reshape(1, 64)
    i2 = lax.broadcasted_iota(jnp.int32, (64, 64), 0)
    j2 = lax.broadcasted_iota(jnp.int32, (64, 64), 1)
    sl = (i2 < j2).astype(jnp.float32)
    ro = jnp.dot(rt, sl, preferred_element_type=jnp.float32)
    return (cs + ro.reshape(64, 1)).reshape(_N_SDR)


def _topk_body(act_ref, cons_ref, sdr_ref, idx_ref):
    c = pl.program_id(0)
    a = act_ref[0, 0, :]
    bits = lax.bitcast_convert_type(a, jnp.uint32)
    u = jnp.where(bits >> 31 != jnp.uint32(0), ~bits,
                  bits | jnp.uint32(0x80000000))

    def srch(i, t):
        cand = t | (jnp.uint32(1) << (31 - i).astype(jnp.uint32))
        cnt = jnp.sum((u >= cand).astype(jnp.int32))
        return jnp.where(cnt >= _W_SPARSITY, cand, t)

    tt = lax.fori_loop(0, 32, srch, jnp.uint32(0))
    gt = u > tt
    g = jnp.sum(gt.astype(jnp.int32))
    eq = u == tt
    rank_eq = _cumsum8192(eq.astype(jnp.float32))
    need = (_W_SPARSITY - g).astype(jnp.float32)
    mask = gt | (eq & (rank_eq <= need))
    sdr = mask.astype(jnp.float32)
    sdr_ref[...] = sdr[None, None, :]

    @pl.when(c == 0)
    def _():
        cons_ref[...] = sdr[None, :]

    @pl.when(c != 0)
    def _():
        cons_ref[...] = cons_ref[...] * sdr[None, :]

    rank_m = _cumsum8192(sdr) * sdr
    jcol = lax.broadcasted_iota(jnp.int32, (_N_SDR, _IDX_PAD), 1).astype(jnp.float32)
    onehot = (rank_m[:, None] == (jcol + 1.0)).astype(jnp.float32)
    iota = lax.broadcasted_iota(jnp.int32, (1, _N_SDR), 1).astype(jnp.float32)
    idx_vals = jnp.dot(iota, onehot, preferred_element_type=jnp.float32)
    idx_ref[...] = idx_vals.astype(jnp.int32)[None]


def _stage_b(act4):
    return pl.pallas_call(
        _topk_body,
        grid=(_N_COLUMNS,),
        in_specs=[pl.BlockSpec((1, 1, _N_SDR), lambda c: (c, 0, 0))],
        out_specs=[
            pl.BlockSpec((1, _N_SDR), lambda c: (0, 0)),
            pl.BlockSpec((1, 1, _N_SDR), lambda c: (c, 0, 0)),
            pl.BlockSpec((1, 1, _IDX_PAD), lambda c: (c, 0, 0)),
        ],
        out_shape=[
            jax.ShapeDtypeStruct((1, _N_SDR), jnp.float32),
            jax.ShapeDtypeStruct((_N_COLUMNS, 1, _N_SDR), jnp.float32),
            jax.ShapeDtypeStruct((_N_COLUMNS, 1, _IDX_PAD), jnp.int32),
        ],
        interpret=_INTERPRET,
    )(act4)


def _ovl_body(perm_ref, sdr_ref, o_ref):
    conn = (perm_ref[...] >= 0.5).astype(jnp.float32)
    o_ref[...] = lax.dot_general(
        conn, sdr_ref[0], (((1,), (1,)), ((), ())),
        preferred_element_type=jnp.float32)


def _stage_c_dense(perm2d, sdr4):
    br = 256
    n = perm2d.shape[0] // br
    blocks_per_col = _N_MINI // br
    return pl.pallas_call(
        _ovl_body,
        grid=(n,),
        in_specs=[
            pl.BlockSpec((br, _N_SDR), lambda i: (i, 0)),
            pl.BlockSpec((1, 1, _N_SDR), lambda i: (i // blocks_per_col, 0, 0)),
        ],
        out_specs=pl.BlockSpec((br, 1), lambda i: (i, 0)),
        out_shape=jax.ShapeDtypeStruct((perm2d.shape[0], 1), jnp.float32),
        interpret=_INTERPRET,
    )(perm2d, sdr4)


def _code_body(ov_ref, wl_ref, wv_ref, b_ref, v_ref, d_ref, out_ref):
    ov = ov_ref[0, 0, :]
    ji = lax.broadcasted_iota(jnp.int32, (1, _N_MINI), 1)[0]
    comp = ov.astype(jnp.int32) * _N_MINI + (_N_MINI - 1 - ji)

    def srch(i, t):
        cand = t | (jnp.int32(1) << (18 - i))
        cnt = jnp.sum((comp >= cand).astype(jnp.int32))
        return jnp.where(cnt >= _K_ACTIVE, cand, t)

    tt = lax.fori_loop(0, 19, srch, jnp.int32(0))
    maskf = (comp >= tt).astype(jnp.float32)

    active = jnp.sum(wl_ref[0] * maskf[None, :], axis=1)
    base = jnp.sum(wv_ref[0] * v_ref[...], axis=1)
    allo = jnp.tanh(active + base + b_ref[0, 0, :])
    ph = jnp.mod(allo / d_ref[0, :], 1.0)
    out_ref[0, 0, :] = jnp.sin(2.0 * jnp.pi * ph)
    out_ref[0, 1, :] = jnp.cos(2.0 * jnp.pi * ph)


def _stage_d(ov4, wl, wv, bpad, vrow, drow):
    return pl.pallas_call(
        _code_body,
        grid=(_N_COLUMNS,),
        in_specs=[
            pl.BlockSpec((1, 1, _N_MINI), lambda c: (c, 0, 0)),
            pl.BlockSpec((1, 16, _N_MINI), lambda c: (c, 0, 0)),
            pl.BlockSpec((1, 16, 128), lambda c: (c, 0, 0)),
            pl.BlockSpec((1, 1, 16), lambda c: (c, 0, 0)),
            pl.BlockSpec((1, 128), lambda c: (0, 0)),
            pl.BlockSpec((1, 16), lambda c: (0, 0)),
        ],
        out_specs=pl.BlockSpec((1, 2, 16), lambda c: (c, 0, 0)),
        out_shape=jax.ShapeDtypeStruct((_N_COLUMNS, 2, 16), jnp.float32),
        interpret=_INTERPRET,
    )(ov4, wl, wv, bpad, vrow, drow)


def kernel(s_t, v_t, W_enc, permanences, W_l6b, b_l6b):
    s_row = s_t.reshape(1, _INPUT_DIM)
    w2d = W_enc.reshape(_N_COLUMNS * _N_SDR, _INPUT_DIM)
    act = _stage_a(w2d, s_row).reshape(_N_COLUMNS, 1, _N_SDR)

    cons, sdr4, idx4 = _stage_b(act)

    perm2d = permanences.reshape(_N_COLUMNS * _N_MINI, _N_SDR)
    ov = _stage_c_dense(perm2d, sdr4).reshape(_N_COLUMNS, 1, _N_MINI)

    wl = jnp.pad(W_l6b[:, :, :_N_MINI], ((0, 0), (0, 4), (0, 0)))
    wv = jnp.pad(W_l6b[:, :, _N_SDR:], ((0, 0), (0, 4), (0, 126)))
    bpad = jnp.pad(b_l6b, ((0, 0), (0, 4))).reshape(_N_COLUMNS, 1, 16)
    vrow = jnp.zeros((1, 128), jnp.float32).at[0, :2].set(v_t)
    periods = np.array([5.0, 7.0, 11.0, 13.0, 17.0, 19.0], np.float32)
    drow = jnp.asarray(
        np.concatenate([np.repeat(periods, 2), np.ones(4, np.float32)])
    ).reshape(1, 16)

    sincos = _stage_d(ov, wl, wv, bpad, vrow, drow)
    sin_p = sincos[:, 0, :12].reshape(_N_COLUMNS, 6, 2)
    cos_p = sincos[:, 1, :12].reshape(_N_COLUMNS, 6, 2)
    codes = jnp.concatenate([sin_p, cos_p], axis=2).reshape(-1)
    return jnp.concatenate([cons.reshape(-1), codes])
